# Initial kernel scaffold; baseline (speedup 1.0000x reference)
import functools

import jax
import jax.numpy as jnp
from jax import lax
from jax.experimental import pallas as pl
from jax.experimental.pallas import tpu as pltpu
from jax.experimental.pallas import tpu_sc as plsc

L = 16
NC = 2
NS = 16
CHUNK = 128
HALF = 128

_MESH = plsc.VectorSubcoreMesh(core_axis_name="c", subcore_axis_name="s")


def _sc_deg(col2d, w2d, n_nodes):
    nchunks = col2d.shape[0]
    cpt = nchunks // (NC * NS)
    rows_per_tile = n_nodes // NS

    @functools.partial(
        pl.kernel,
        out_type=jax.ShapeDtypeStruct((NC * n_nodes, L), jnp.float32),
        mesh=_MESH,
        scratch_types=[
            pltpu.VMEM_SHARED((n_nodes, L), jnp.float32),
            pltpu.VMEM((1, CHUNK), jnp.int32),
            pltpu.VMEM((CHUNK,), jnp.float32),
            pltpu.VMEM((CHUNK, L), jnp.float32),
            pltpu.VMEM((n_nodes // NS, L), jnp.float32),
        ],
    )
    def deg_kernel(col_hbm, w_hbm, degp_hbm, deg_sp, coli, wv, src, zbuf):
        cid = lax.axis_index("c")
        sid = lax.axis_index("s")
        row0 = sid * rows_per_tile
        zero = jnp.zeros((L,), jnp.float32)

        @pl.loop(0, rows_per_tile)
        def _(r):
            zbuf[r, :] = zero

        pltpu.sync_copy(zbuf, deg_sp.at[pl.ds(row0, rows_per_tile)])
        plsc.subcore_barrier()

        base = (cid * NS + sid) * cpt

        @pl.loop(0, cpt)
        def _(k):
            j = base + k
            pltpu.sync_copy(col_hbm.at[j], coli.at[0])
            pltpu.sync_copy(w_hbm.at[j], wv)

            @pl.loop(0, CHUNK)
            def _(e):
                idx = jnp.full((L,), e, dtype=jnp.int32)
                src[e, :] = plsc.load_gather(wv, [idx])

            pltpu.sync_copy(src, deg_sp.at[coli.at[0]], add=True)

        plsc.subcore_barrier()
        pltpu.sync_copy(
            deg_sp.at[pl.ds(row0, rows_per_tile)],
            degp_hbm.at[pl.ds(cid * n_nodes + row0, rows_per_tile)],
        )

    return deg_kernel(col2d, w2d)


def _sc_msg(y_flat, row2d, col2d, w2d, n_nodes):
    nchunks = row2d.shape[0]
    cpt = nchunks // NS
    rows_per_tile = n_nodes // NS

    @functools.partial(
        pl.kernel,
        out_type=jax.ShapeDtypeStruct((NC * n_nodes, HALF), jnp.float32),
        mesh=_MESH,
        scratch_types=[
            pltpu.VMEM_SHARED((n_nodes, HALF), jnp.float32),
            pltpu.VMEM((1, CHUNK), jnp.int32),
            pltpu.VMEM((1, CHUNK), jnp.int32),
            pltpu.VMEM((CHUNK,), jnp.float32),
            pltpu.VMEM((CHUNK, HALF), jnp.float32),
        ],
    )
    def msg_kernel(y_hbm, row_hbm, col_hbm, out_hbm, acc_sp, rowi, coli, wv, buf):
        cid = lax.axis_index("c")
        sid = lax.axis_index("s")
        row0 = sid * rows_per_tile
        slab0 = cid * n_nodes

        pltpu.sync_copy(
            y_hbm.at[pl.ds(slab0 + row0, rows_per_tile)],
            acc_sp.at[pl.ds(row0, rows_per_tile)],
        )
        plsc.subcore_barrier()

        base = sid * cpt
        offv = jnp.full((L,), slab0, dtype=jnp.int32)

        @pl.loop(0, cpt)
        def _(k):
            j = base + k
            pltpu.sync_copy(row_hbm.at[j], rowi.at[0])
            pltpu.sync_copy(col_hbm.at[j], coli.at[0])
            pltpu.sync_copy(w_hbm.at[j], wv)
            for f in range(CHUNK // L):
                sl = pl.ds(f * L, L)
                rowi[0, sl] = rowi[0, sl] + offv

            pltpu.sync_copy(y_hbm.at[rowi.at[0]], buf)

            @pl.loop(0, CHUNK)
            def _(e):
                idx = jnp.full((L,), e, dtype=jnp.int32)
                wsp = plsc.load_gather(wv, [idx])
                for f in range(HALF // L):
                    sl = pl.ds(f * L, L)
                    buf[e, sl] = buf[e, sl] * wsp

            pltpu.sync_copy(buf, acc_sp.at[coli.at[0]], add=True)

        plsc.subcore_barrier()
        pltpu.sync_copy(
            acc_sp.at[pl.ds(row0, rows_per_tile)],
            out_hbm.at[pl.ds(slab0 + row0, rows_per_tile)],
        )

    return msg_kernel(y_flat, row2d, col2d, w2d)


def _dis_from_degp(degp_blk):
    deg = degp_blk[0, :, 0] + degp_blk[1, :, 0] + 1.0
    return jnp.where(deg > 0, lax.rsqrt(jnp.maximum(deg, 1e-12)), 0.0)


def _tc_y(x, W, degp3):
    n, d_in = x.shape
    d_out = W.shape[0]
    br = n // 10

    def body(x_ref, w_ref, degp_ref, y_ref):
        xl = lax.dot_general(
            x_ref[...], w_ref[...], (((1,), (1,)), ((), ())),
            preferred_element_type=jnp.float32,
            precision=lax.Precision.HIGHEST,
        )
        y = xl * _dis_from_degp(degp_ref)[:, None]
        y_ref[...] = jnp.stack([y[:, :HALF], y[:, HALF:]], axis=0)

    return pl.pallas_call(
        body,
        grid=(n // br,),
        in_specs=[
            pl.BlockSpec((br, d_in), lambda i: (i, 0)),
            pl.BlockSpec((d_out, d_in), lambda i: (0, 0)),
            pl.BlockSpec((NC, br, L), lambda i: (0, i, 0)),
        ],
        out_specs=pl.BlockSpec((NC, br, HALF), lambda i: (0, i, 0)),
        out_shape=jax.ShapeDtypeStruct((NC, n, HALF), jnp.float32),
    )(x, W, degp3)


def _tc_final(acc3, degp3, b2d):
    n = acc3.shape[1]
    d_out = NC * HALF
    br = n // 10

    def body(acc_ref, degp_ref, b_ref, o_ref):
        dis = _dis_from_degp(degp_ref)
        m = jnp.concatenate([acc_ref[0], acc_ref[1]], axis=1)
        o_ref[...] = m * dis[:, None] + b_ref[...]

    return pl.pallas_call(
        body,
        grid=(n // br,),
        in_specs=[
            pl.BlockSpec((NC, br, HALF), lambda i: (0, i, 0)),
            pl.BlockSpec((NC, br, L), lambda i: (0, i, 0)),
            pl.BlockSpec((1, d_out), lambda i: (0, 0)),
        ],
        out_specs=pl.BlockSpec((br, d_out), lambda i: (i, 0)),
        out_shape=jax.ShapeDtypeStruct((n, d_out), jnp.float32),
    )(acc3, degp3, b2d)


def kernel(x, edge_index, edge_weight, W, b):
    n = x.shape[0]
    e = edge_weight.shape[0]
    row = edge_index[0].astype(jnp.int32)
    col = edge_index[1].astype(jnp.int32)
    w = edge_weight.astype(jnp.float32)

    nchunks = -(-e // CHUNK)
    nchunks = -(-nchunks // (NC * NS)) * (NC * NS)
    pad = nchunks * CHUNK - e
    pad_idx = (jnp.arange(pad, dtype=jnp.int32) * 37) % n
    row2d = jnp.concatenate([row, pad_idx]).reshape(nchunks, CHUNK)
    col2d = jnp.concatenate([col, pad_idx]).reshape(nchunks, CHUNK)
    w2d = jnp.concatenate([w, jnp.zeros((pad,), jnp.float32)]).reshape(
        nchunks, CHUNK)

    degp = _sc_deg(col2d, w2d, n)
    degp3 = degp.reshape(NC, n, L)
    y = _tc_y(x, W, degp3)
    acc = _sc_msg(y.reshape(NC * n, HALF), row2d, col2d, w2d, n)
    return _tc_final(acc.reshape(NC, n, HALF), degp3, b.reshape(1, NC * HALF))


# speedup vs baseline: 8.4637x; 8.4637x over previous
import functools

import jax
import jax.numpy as jnp
from jax import lax
from jax.experimental import pallas as pl
from jax.experimental.pallas import tpu as pltpu
from jax.experimental.pallas import tpu_sc as plsc

L = 16
NC = 2
NS = 16
CHUNK = 128
HALF = 128

_MESH = plsc.VectorSubcoreMesh(core_axis_name="c", subcore_axis_name="s")
_SC_PARAMS = pltpu.CompilerParams(needs_layout_passes=False)


def _sc_deg(col3d, w3d, npad):
    nchunks = col3d.shape[0]
    cpt = nchunks // (NC * NS)

    @functools.partial(
        pl.kernel,
        out_type=jax.ShapeDtypeStruct((NC * NS * npad,), jnp.float32),
        mesh=_MESH,
        compiler_params=_SC_PARAMS,
        scratch_types=[
            pltpu.VMEM((npad,), jnp.float32),
            pltpu.VMEM((1, CHUNK), jnp.int32),
            pltpu.VMEM((1, CHUNK), jnp.float32),
        ],
    )
    def deg_kernel(col_hbm, w_hbm, degp_hbm, deg_v, coli, wv):
        cid = lax.axis_index("c")
        sid = lax.axis_index("s")
        wid = cid * NS + sid
        zero = jnp.zeros((L,), jnp.float32)

        @pl.loop(0, npad // L)
        def _(g):
            deg_v[pl.ds(g * L, L)] = zero

        base = wid * cpt

        @pl.loop(0, cpt)
        def _(k):
            j = base + k
            pltpu.sync_copy(col_hbm.at[j], coli)
            pltpu.sync_copy(w_hbm.at[j], wv)
            for g in range(CHUNK // L):
                sl = pl.ds(g * L, L)
                plsc.addupdate_scatter(deg_v, [coli[0, sl]], wv[0, sl])

        pltpu.sync_copy(deg_v, degp_hbm.at[pl.ds(wid * npad, npad)])

    return deg_kernel(col3d, w3d)


def _sc_msg(y_flat, row3d, col3d, w3d, npad):
    nchunks = row3d.shape[0]
    cpt = nchunks // NS
    rpt = npad // NS

    @functools.partial(
        pl.kernel,
        out_type=jax.ShapeDtypeStruct((NC * npad, HALF), jnp.float32),
        mesh=_MESH,
        compiler_params=_SC_PARAMS,
        scratch_types=[
            pltpu.VMEM_SHARED((npad, HALF), jnp.float32),
            pltpu.VMEM((1, CHUNK), jnp.int32),
            pltpu.VMEM((1, CHUNK), jnp.int32),
            pltpu.VMEM((1, CHUNK), jnp.float32),
            pltpu.VMEM((CHUNK, HALF), jnp.float32),
        ],
    )
    def msg_kernel(y_hbm, row_hbm, col_hbm, w_hbm, out_hbm,
                   acc_sp, rowi, coli, wv, buf):
        cid = lax.axis_index("c")
        sid = lax.axis_index("s")
        row0 = sid * rpt
        slab0 = cid * npad
        zrow = jnp.zeros((L,), jnp.int32)

        pltpu.sync_copy(
            y_hbm.at[pl.ds(slab0 + row0, rpt)],
            acc_sp.at[pl.ds(row0, rpt)],
        )
        plsc.subcore_barrier()

        base = sid * cpt
        offv = jnp.full((L,), slab0, dtype=jnp.int32)

        @pl.loop(0, cpt)
        def _(k):
            j = base + k
            pltpu.sync_copy(row_hbm.at[j], rowi)
            pltpu.sync_copy(col_hbm.at[j], coli)
            pltpu.sync_copy(w_hbm.at[j], wv)
            for f in range(CHUNK // L):
                sl = pl.ds(f * L, L)
                rowi[0, sl] = rowi[0, sl] + offv

            pltpu.sync_copy(y_hbm.at[rowi.at[0]], buf)

            @pl.loop(0, CHUNK)
            def _(e):
                idx = jnp.full((L,), e, dtype=jnp.int32)
                wsp = plsc.load_gather(wv, [zrow, idx])
                for f in range(HALF // L):
                    sl = pl.ds(f * L, L)
                    buf[e, sl] = buf[e, sl] * wsp

            pltpu.sync_copy(buf, acc_sp.at[coli.at[0]], add=True)

        plsc.subcore_barrier()
        pltpu.sync_copy(
            acc_sp.at[pl.ds(row0, rpt)],
            out_hbm.at[pl.ds(slab0 + row0, rpt)],
        )

    return msg_kernel(y_flat, row3d, col3d, w3d)


def _dis_from_degp(degp_blk):
    deg = jnp.sum(degp_blk, axis=0) + 1.0
    return jnp.where(deg > 0, lax.rsqrt(jnp.maximum(deg, 1e-12)), 0.0)


def _tc_y(x_pad, W, degp2):
    npad, d_in = x_pad.shape
    d_out = W.shape[0]
    br = npad // 8

    def body(x_ref, w_ref, degp_ref, y_ref):
        xl = lax.dot_general(
            x_ref[...], w_ref[...], (((1,), (1,)), ((), ())),
            preferred_element_type=jnp.float32,
            precision=lax.Precision.HIGHEST,
        )
        y = xl * _dis_from_degp(degp_ref[...])[:, None]
        y_ref[...] = jnp.stack([y[:, :HALF], y[:, HALF:]], axis=0)

    return pl.pallas_call(
        body,
        grid=(npad // br,),
        in_specs=[
            pl.BlockSpec((br, d_in), lambda i: (i, 0)),
            pl.BlockSpec((d_out, d_in), lambda i: (0, 0)),
            pl.BlockSpec((NC * NS, br), lambda i: (0, i)),
        ],
        out_specs=pl.BlockSpec((NC, br, HALF), lambda i: (0, i, 0)),
        out_shape=jax.ShapeDtypeStruct((NC, npad, HALF), jnp.float32),
    )(x_pad, W, degp2)


def _tc_final(acc3, degp2, b2d):
    npad = acc3.shape[1]
    d_out = NC * HALF
    br = npad // 10

    def body(acc_ref, degp_ref, b_ref, o_ref):
        dis = _dis_from_degp(degp_ref[...])
        m = jnp.concatenate([acc_ref[0], acc_ref[1]], axis=1)
        o_ref[...] = m * dis[:, None] + b_ref[...]

    return pl.pallas_call(
        body,
        grid=(npad // br,),
        in_specs=[
            pl.BlockSpec((NC, br, HALF), lambda i: (0, i, 0)),
            pl.BlockSpec((NC * NS, br), lambda i: (0, i)),
            pl.BlockSpec((1, d_out), lambda i: (0, 0)),
        ],
        out_specs=pl.BlockSpec((br, d_out), lambda i: (i, 0)),
        out_shape=jax.ShapeDtypeStruct((npad, d_out), jnp.float32),
    )(acc3, degp2, b2d)


def kernel(x, edge_index, edge_weight, W, b):
    n = x.shape[0]
    e = edge_weight.shape[0]
    npad = -(-n // (NS * L)) * (NS * L)
    row = edge_index[0].astype(jnp.int32)
    col = edge_index[1].astype(jnp.int32)
    w = edge_weight.astype(jnp.float32)

    nchunks = -(-e // CHUNK)
    nchunks = -(-nchunks // (NC * NS)) * (NC * NS)
    pad = nchunks * CHUNK - e
    pad_idx = (jnp.arange(pad, dtype=jnp.int32) * 37) % n
    row3d = jnp.concatenate([row, pad_idx]).reshape(nchunks, 1, CHUNK)
    col3d = jnp.concatenate([col, pad_idx]).reshape(nchunks, 1, CHUNK)
    w3d = jnp.concatenate([w, jnp.zeros((pad,), jnp.float32)]).reshape(
        nchunks, 1, CHUNK)
    x_pad = jnp.concatenate(
        [x, jnp.zeros((npad - n, x.shape[1]), x.dtype)], axis=0)

    degp = _sc_deg(col3d, w3d, npad)
    degp2 = degp.reshape(NC * NS, npad)
    y = _tc_y(x_pad, W, degp2)
    acc = _sc_msg(y.reshape(NC * npad, HALF), row3d, col3d, w3d, npad)
    out = _tc_final(acc.reshape(NC, npad, HALF), degp2,
                    b.reshape(1, NC * HALF))
    return out[:n]
